# baseline (speedup 1.0000x reference)
import functools

import jax
import jax.numpy as jnp
from jax import lax
from jax.experimental import pallas as pl
from jax.experimental.pallas import tpu as pltpu
from jax.experimental.pallas import tpu_sc as plsc

_NE = 1024
_D = 128
_N = 32 * 1024
_BT = 512
_SCALE = 1.25 / (_N * _D)
_NHALF = _N // 2

_NW = 32
_CH = 128


def _tc_body(x_ref, cbm2_ref, cbn_ref, ids_ref, idx_ref, loss_ref):
    i = pl.program_id(0)
    xb = x_ref[...]
    cbm2 = cbm2_ref[...]
    ab2 = lax.dot_general(cbm2, xb, (((1,), (1,)), ((), ())),
                          preferred_element_type=jnp.float32)
    xsqT = (xb * xb).T
    s = xsqT[0:8, :]
    for j in range(1, 16):
        s = s + xsqT[8 * j:8 * (j + 1), :]
    w = 8
    while w > 1:
        w //= 2
        s = s[:w, :] + s[w:, :]
    rnT = s
    cbnT = cbn_ref[...]
    d2 = (rnT + ab2) + cbnT
    dist = jnp.sqrt(jnp.maximum(d2, 0.0))
    minv = jnp.min(dist, axis=0, keepdims=True)
    ids = ids_ref[...]
    idxf = jnp.min(jnp.where(dist == minv, ids, jnp.float32(_NE)), axis=0)
    idx_ref[0, 0, :] = idxf.astype(jnp.int32)

    mv = minv[0, :]
    part = jnp.sum(mv * mv)

    @pl.when(i == 0)
    def _init():
        loss_ref[0, 0] = 0.0

    loss_ref[0, 0] += part


def _make_tc(ntok):
    grid = ntok // _BT
    return pl.pallas_call(
        _tc_body,
        grid=(grid,),
        in_specs=[
            pl.BlockSpec((_BT, _D), lambda i: (i, 0)),
            pl.BlockSpec((_NE, _D), lambda i: (0, 0)),
            pl.BlockSpec((_NE, 1), lambda i: (0, 0)),
            pl.BlockSpec((_NE, _BT), lambda i: (0, 0)),
        ],
        out_specs=[
            pl.BlockSpec((1, 1, _BT), lambda i: (i, 0, 0)),
            pl.BlockSpec(memory_space=pltpu.SMEM),
        ],
        out_shape=[
            jax.ShapeDtypeStruct((grid, 1, _BT), jnp.int32),
            jax.ShapeDtypeStruct((1, 1), jnp.float32),
        ],
    )


_tc_half = _make_tc(_NHALF)


def _sc_gather(codebook, idx2d, nrows):
    rpw = nrows // _NW
    nch = rpw // _CH
    mesh = plsc.VectorSubcoreMesh(core_axis_name="c", subcore_axis_name="s")

    @functools.partial(
        pl.kernel,
        mesh=mesh,
        out_type=jax.ShapeDtypeStruct((nrows, _D), jnp.float32),
        scratch_types=[
            pltpu.VMEM((nch, _CH), jnp.int32),
            pltpu.VMEM((_CH, _D), jnp.float32),
            pltpu.VMEM((_CH, _D), jnp.float32),
            pltpu.SemaphoreType.DMA,
            pltpu.SemaphoreType.DMA,
        ],
    )
    def k(cb_hbm, idx_hbm, out_hbm, idx_v, r0, r1, s0, s1):
        wid = lax.axis_index("s") * 2 + lax.axis_index("c")
        pltpu.sync_copy(idx_hbm.at[pl.ds(wid * nch, nch)], idx_v)
        bufs = (r0, r1)
        sems = (s0, s1)
        copies = [pltpu.async_copy(cb_hbm.at[idx_v.at[0]], r0, s0)]
        for j in range(nch):
            if j + 1 < nch:
                copies.append(pltpu.async_copy(
                    cb_hbm.at[idx_v.at[j + 1]], bufs[(j + 1) % 2],
                    sems[(j + 1) % 2]))
            copies[j].wait()
            pltpu.sync_copy(bufs[j % 2],
                            out_hbm.at[pl.ds(wid * rpw + j * _CH, _CH)])

    return k(codebook, idx2d)


def kernel(x, codebook):
    flat = x.reshape(_N, _D)
    cbm2 = -2.0 * codebook
    cbn = jnp.sum(codebook * codebook, axis=1)[:, None]
    ids = lax.broadcasted_iota(jnp.float32, (_NE, _BT), 0)
    idx3_a, acc_a = _tc_half(flat[:_NHALF], cbm2, cbn, ids)
    ind_a = idx3_a.reshape(_NHALF)
    q_a = _sc_gather(codebook, ind_a.reshape(_NHALF // _D, _D), _NHALF)
    idx3_b, acc_b = _tc_half(flat[_NHALF:], cbm2, cbn, ids)
    ind_b = idx3_b.reshape(_NHALF)
    q_b = _sc_gather(codebook, ind_b.reshape(_NHALF // _D, _D), _NHALF)
    loss = (acc_a[0, 0] + acc_b[0, 0]) * _SCALE
    indices = jnp.concatenate([ind_a, ind_b])
    q = jnp.concatenate([q_a, q_b]).reshape(x.shape)
    return q, loss, indices


# speedup vs baseline: 1.2176x; 1.2176x over previous
import functools

import jax
import jax.numpy as jnp
from jax import lax
from jax.experimental import pallas as pl
from jax.experimental.pallas import tpu as pltpu
from jax.experimental.pallas import tpu_sc as plsc

_NE = 1024
_D = 128
_N = 32 * 1024
_BT = 512
_SCALE = 1.25 / (_N * _D)
_NHALF = _N // 2

_NW = 32
_CH = 128


def _tc_body(x_ref, cbm2_ref, cbn_ref, ids_ref, idx_ref, loss_ref):
    i = pl.program_id(0)
    xb = x_ref[...]
    cbm2 = cbm2_ref[...]
    ab2 = lax.dot_general(cbm2, xb, (((1,), (1,)), ((), ())),
                          preferred_element_type=jnp.float32)
    xsqT = (xb * xb).T
    s = xsqT[0:8, :]
    for j in range(1, 16):
        s = s + xsqT[8 * j:8 * (j + 1), :]
    w = 8
    while w > 1:
        w //= 2
        s = s[:w, :] + s[w:, :]
    rnT = s
    cbnT = cbn_ref[...]
    d2 = (rnT + ab2) + cbnT
    dist = jnp.sqrt(jnp.maximum(d2, 0.0))
    minv = jnp.min(dist, axis=0, keepdims=True)
    ids = ids_ref[...]
    idxf = jnp.min(jnp.where(dist == minv, ids, jnp.float32(_NE)), axis=0)
    idx_ref[0, 0, :] = idxf.astype(jnp.int32)

    mv = minv[0, :]
    part = jnp.sum(mv * mv)

    @pl.when(i == 0)
    def _init():
        loss_ref[0, 0] = 0.0

    loss_ref[0, 0] += part


def _make_tc(ntok):
    grid = ntok // _BT
    return pl.pallas_call(
        _tc_body,
        grid=(grid,),
        in_specs=[
            pl.BlockSpec((_BT, _D), lambda i: (i, 0)),
            pl.BlockSpec((_NE, _D), lambda i: (0, 0)),
            pl.BlockSpec((_NE, 1), lambda i: (0, 0)),
            pl.BlockSpec((_NE, _BT), lambda i: (0, 0)),
        ],
        out_specs=[
            pl.BlockSpec((1, 1, _BT), lambda i: (i, 0, 0)),
            pl.BlockSpec(memory_space=pltpu.SMEM),
        ],
        out_shape=[
            jax.ShapeDtypeStruct((grid, 1, _BT), jnp.int32),
            jax.ShapeDtypeStruct((1, 1), jnp.float32),
        ],
    )


_tc_full = _make_tc(_N)


def _sc_gather(codebook, idx2d, nrows):
    rpw = nrows // _NW
    nch = rpw // _CH
    mesh = plsc.VectorSubcoreMesh(core_axis_name="c", subcore_axis_name="s")

    @functools.partial(
        pl.kernel,
        mesh=mesh,
        out_type=jax.ShapeDtypeStruct((nrows, _D), jnp.float32),
        scratch_types=[
            pltpu.VMEM((nch, _CH), jnp.int32),
            pltpu.VMEM((_CH, _D), jnp.float32),
            pltpu.VMEM((_CH, _D), jnp.float32),
            pltpu.SemaphoreType.DMA,
            pltpu.SemaphoreType.DMA,
        ],
    )
    def k(cb_hbm, idx_hbm, out_hbm, idx_v, r0, r1, s0, s1):
        wid = lax.axis_index("s") * 2 + lax.axis_index("c")
        pltpu.sync_copy(idx_hbm.at[pl.ds(wid * nch, nch)], idx_v)
        bufs = (r0, r1)
        sems = (s0, s1)
        copies = [pltpu.async_copy(cb_hbm.at[idx_v.at[0]], r0, s0)]
        for j in range(nch):
            if j + 1 < nch:
                copies.append(pltpu.async_copy(
                    cb_hbm.at[idx_v.at[j + 1]], bufs[(j + 1) % 2],
                    sems[(j + 1) % 2]))
            copies[j].wait()
            pltpu.sync_copy(bufs[j % 2],
                            out_hbm.at[pl.ds(wid * rpw + j * _CH, _CH)])

    return k(codebook, idx2d)


def kernel(x, codebook):
    flat = x.reshape(_N, _D)
    cbm2 = -2.0 * codebook
    cbn = jnp.sum(codebook * codebook, axis=1)[:, None]
    ids = lax.broadcasted_iota(jnp.float32, (_NE, _BT), 0)
    idx3, acc = _tc_full(flat, cbm2, cbn, ids)
    indices = idx3.reshape(_N)
    q = _sc_gather(codebook, indices.reshape(_N // _D, _D), _N)
    loss = acc[0, 0] * _SCALE
    return q.reshape(x.shape), loss, indices


# speedup vs baseline: 1.2348x; 1.0141x over previous
import functools

import jax
import jax.numpy as jnp
import numpy as np
from jax import lax
from jax.experimental import pallas as pl
from jax.experimental.pallas import tpu as pltpu
from jax.experimental.pallas import tpu_sc as plsc

_NE = 1024
_D = 128
_N = 32 * 1024
_BT = 512
_SCALE = 1.25 / (_N * _D)
_NHALF = _N // 2

_NW = 32
_CH = 128
_NBUF = 6

_IDS = np.broadcast_to(
    np.arange(_NE, dtype=np.float32)[:, None], (_NE, _BT)).copy()


def _tc_body(x_ref, cbm2_ref, cbn_ref, ids_ref, idx_ref, loss_ref):
    i = pl.program_id(0)
    xb = x_ref[...]
    cbm2 = cbm2_ref[...]
    ab2 = lax.dot_general(cbm2, xb, (((1,), (1,)), ((), ())),
                          preferred_element_type=jnp.float32)
    xsqT = (xb * xb).T
    s = xsqT[0:8, :]
    for j in range(1, 16):
        s = s + xsqT[8 * j:8 * (j + 1), :]
    w = 8
    while w > 1:
        w //= 2
        s = s[:w, :] + s[w:, :]
    rnT = s
    cbnT = cbn_ref[...]
    d2 = (rnT + ab2) + cbnT
    dist = jnp.sqrt(jnp.maximum(d2, 0.0))
    minv = jnp.min(dist, axis=0, keepdims=True)
    ids = ids_ref[...]
    idxf = jnp.min(jnp.where(dist == minv, ids, jnp.float32(_NE)), axis=0)
    idx_ref[0, 0, :] = idxf.astype(jnp.int32)

    mv = minv[0, :]
    part = jnp.sum(mv * mv)

    @pl.when(i == 0)
    def _init():
        loss_ref[0, 0] = 0.0

    loss_ref[0, 0] += part


def _make_tc(ntok):
    grid = ntok // _BT
    return pl.pallas_call(
        _tc_body,
        grid=(grid,),
        in_specs=[
            pl.BlockSpec((_BT, _D), lambda i: (i, 0)),
            pl.BlockSpec((_NE, _D), lambda i: (0, 0)),
            pl.BlockSpec((_NE, 1), lambda i: (0, 0)),
            pl.BlockSpec((_NE, _BT), lambda i: (0, 0)),
        ],
        out_specs=[
            pl.BlockSpec((1, 1, _BT), lambda i: (i, 0, 0)),
            pl.BlockSpec(memory_space=pltpu.SMEM),
        ],
        out_shape=[
            jax.ShapeDtypeStruct((grid, 1, _BT), jnp.int32),
            jax.ShapeDtypeStruct((1, 1), jnp.float32),
        ],
    )


_tc_full = _make_tc(_N)


def _sc_gather(codebook, idx2d, nrows):
    rpw = nrows // _NW
    nch = rpw // _CH
    mesh = plsc.VectorSubcoreMesh(core_axis_name="c", subcore_axis_name="s")

    @functools.partial(
        pl.kernel,
        mesh=mesh,
        out_type=jax.ShapeDtypeStruct((nrows, _D), jnp.float32),
        scratch_types=(
            [pltpu.VMEM((nch, _CH), jnp.int32)]
            + [pltpu.VMEM((_CH, _D), jnp.float32) for _ in range(_NBUF)]
            + [pltpu.SemaphoreType.DMA for _ in range(2 * _NBUF)]
        ),
    )
    def k(cb_hbm, idx_hbm, out_hbm, idx_v, *rs):
        bufs = rs[:_NBUF]
        gsem = rs[_NBUF:2 * _NBUF]
        wsem = rs[2 * _NBUF:]
        wid = lax.axis_index("s") * 2 + lax.axis_index("c")
        pltpu.sync_copy(idx_hbm.at[pl.ds(wid * nch, nch)], idx_v)
        gathers = [None] * nch
        writes = [None] * nch
        for j in range(min(_NBUF, nch)):
            gathers[j] = pltpu.async_copy(
                cb_hbm.at[idx_v.at[j]], bufs[j % _NBUF], gsem[j % _NBUF])
        for j in range(nch):
            gathers[j].wait()
            writes[j] = pltpu.async_copy(
                bufs[j % _NBUF],
                out_hbm.at[pl.ds(wid * rpw + j * _CH, _CH)],
                wsem[j % _NBUF])
            jp = j - 1
            if jp >= 0 and jp + _NBUF < nch:
                writes[jp].wait()
                gathers[jp + _NBUF] = pltpu.async_copy(
                    cb_hbm.at[idx_v.at[jp + _NBUF]], bufs[jp % _NBUF],
                    gsem[jp % _NBUF])
        for j in range(max(nch - _NBUF, 0), nch):
            writes[j].wait()

    return k(codebook, idx2d)


def kernel(x, codebook):
    flat = x.reshape(_N, _D)
    cbm2 = -2.0 * codebook
    cbn = jnp.sum(codebook * codebook, axis=1)[:, None]
    idx3, acc = _tc_full(flat, cbm2, cbn, _IDS)
    indices = idx3.reshape(_N)
    q = _sc_gather(codebook, indices.reshape(_N // _D, _D), _N)
    loss = acc[0, 0] * _SCALE
    return q.reshape(x.shape), loss, indices
